# baseline (speedup 1.0000x reference)
import functools
import numpy as np
import jax
import jax.numpy as jnp
from jax.experimental import pallas as pl

HI = jax.lax.Precision.HIGHEST


def _conv_body(x_ref, w_ref, b_ref, o_ref, pad_ref, *, bn=None):
    Bt, S, _, C = x_ref.shape
    Co = w_ref.shape[2]
    pad_ref[...] = jnp.zeros_like(pad_ref)
    pad_ref[:, 1:S + 1, 1:S + 1, :] = x_ref[...]
    acc = None
    for dh in range(3):
        for dw in range(3):
            xs = pad_ref[:, dh:dh + S, dw:dw + S, :]
            t = jnp.dot(xs.reshape(Bt * S * S, C), w_ref[3 * dh + dw],
                        preferred_element_type=jnp.float32)
            acc = t if acc is None else acc + t
    acc = jnp.maximum(acc + b_ref[...], 0.0)
    y = acc.reshape(Bt, S // 2, 2, S, Co)
    y = jnp.max(y, axis=2)
    y = y.reshape(Bt, S // 2, S // 2, 2, Co)
    y = jnp.max(y, axis=3)
    if bn is not None:
        g_ref, be_ref, mu_ref, va_ref = bn
        scale = g_ref[...] * jax.lax.rsqrt(va_ref[...] + 1e-5)
        shift = be_ref[...] - mu_ref[...] * scale
        y = y * scale[None, None] + shift[None, None]
    o_ref[...] = y


def _conv_stage(x, w9, b, Bt, bn=None):
    B, S, _, C = x.shape
    Co = w9.shape[2]
    n = B // Bt
    full = lambda a: pl.BlockSpec(a.shape, lambda i: (0,) * a.ndim)
    in_specs = [pl.BlockSpec((Bt, S, S, C), lambda i: (i, 0, 0, 0)),
                full(w9), full(b)]
    args = [x, w9, b]
    body = _conv_body
    if bn is not None:
        in_specs += [full(p) for p in bn]
        args += list(bn)

        def body(x_ref, w_ref, b_ref, g, be, mu, va, o_ref, pad_ref):
            _conv_body(x_ref, w_ref, b_ref, o_ref, pad_ref,
                       bn=(g, be, mu, va))
    return pl.pallas_call(
        body,
        grid=(n,),
        in_specs=in_specs,
        out_specs=pl.BlockSpec((Bt, S // 2, S // 2, Co),
                               lambda i: (i, 0, 0, 0)),
        out_shape=jax.ShapeDtypeStruct((B, S // 2, S // 2, Co), jnp.float32),
        scratch_shapes=[pltpu_vmem((Bt, S + 2, S + 2, C))],
    )(*args)


def pltpu_vmem(shape):
    from jax.experimental.pallas import tpu as pltpu
    return pltpu.VMEM(shape, jnp.float32)


def _onehot_argmax(P):
    m = jnp.max(P, axis=1, keepdims=True)
    iota = jax.lax.broadcasted_iota(jnp.int32, P.shape, 1)
    cand = jnp.where(P >= m, iota, P.shape[1])
    idx = jnp.min(cand, axis=1, keepdims=True)
    return (iota == idx).astype(jnp.float32)


def _expert_body(y_ref, t_ref, p1_ref, p2_ref, p3_ref,
                 w1_ref, b1_ref, w2_ref, b2_ref, w3_ref, b3_ref, o_ref):
    T = t_ref[...]
    y = y_ref[...]
    NM = w1_ref.shape[0]
    for pol, W, bb in ((p1_ref, w1_ref, b1_ref), (p2_ref, w2_ref, b2_ref),
                       (p3_ref, w3_ref, b3_ref)):
        P = jnp.dot(T, pol[...], precision=HI,
                    preferred_element_type=jnp.float32)
        mask = _onehot_argmax(P)
        acc = None
        for e in range(NM):
            t = mask[:, e:e + 1] * jnp.dot(y, W[e],
                                           preferred_element_type=jnp.float32)
            acc = t if acc is None else acc + t
        y = acc + jnp.dot(mask, bb[...], precision=HI,
                          preferred_element_type=jnp.float32)
    o_ref[...] = y[:, :o_ref.shape[1]]


def _experts(yflat, Toh, pol1, pol2, pol3, W1p, b1, W2, b2, W3p, b3p, Bt):
    B = yflat.shape[0]
    OUT = 10
    n = B // Bt
    full = lambda a: pl.BlockSpec(a.shape, lambda i: (0,) * a.ndim)
    return pl.pallas_call(
        _expert_body,
        grid=(n,),
        in_specs=[pl.BlockSpec((Bt, yflat.shape[1]), lambda i: (i, 0)),
                  pl.BlockSpec((Bt, Toh.shape[1]), lambda i: (i, 0)),
                  full(pol1), full(pol2), full(pol3),
                  full(W1p), full(b1), full(W2), full(b2),
                  full(W3p), full(b3p)],
        out_specs=pl.BlockSpec((Bt, OUT), lambda i: (i, 0)),
        out_shape=jax.ShapeDtypeStruct((B, OUT), jnp.float32),
    )(yflat, Toh, pol1, pol2, pol3, W1p, b1, W2, b2, W3p, b3p)


def kernel(x, tasks, conv_w1, conv_b1, conv_w2, conv_b2, conv_w3, conv_b3,
           bn_gamma, bn_beta, bn_mean, bn_var, policy1, policy2, policy3,
           W1, b1, W2, b2, W3, b3):
    B = x.shape[0]
    NM, FLAT, HID = W1.shape
    NA = policy1.shape[0]

    xh = jnp.transpose(x, (0, 2, 3, 1))
    w9 = lambda w: jnp.transpose(w, (2, 3, 1, 0)).reshape(9, w.shape[1],
                                                          w.shape[0])
    row = lambda v: v.reshape(1, -1)
    j = np.arange(FLAT)
    perm = (j % 32) * 16 + (j // 128) * 4 + ((j // 32) % 4)
    W1p = W1[:, perm, :]
    W3p = jnp.pad(W3, ((0, 0), (0, 0), (0, 128 - W3.shape[2])))
    b3p = jnp.pad(b3, ((0, 0), (0, 128 - b3.shape[1])))
    Toh = (tasks[:, None] == jnp.arange(NA)[None, :]).astype(jnp.float32)

    y = _conv_stage(xh, w9(conv_w1), row(conv_b1), Bt=8)
    y = _conv_stage(y, w9(conv_w2), row(conv_b2), Bt=32)
    y = _conv_stage(y, w9(conv_w3), row(conv_b3), Bt=64,
                    bn=(row(bn_gamma), row(bn_beta), row(bn_mean),
                        row(bn_var)))
    yflat = y.reshape(B, FLAT)
    return _experts(yflat, Toh, policy1, policy2, policy3,
                    W1p, b1, W2, b2, W3p, b3p, Bt=128)
